# baseline (speedup 1.0000x reference)
import jax
import jax.numpy as jnp
from jax import lax
from jax.experimental import pallas as pl
from jax.experimental.pallas import tpu as pltpu
from jax.experimental.pallas import tpu_sc as plsc

N = 10000
E = 320000
D = 128
D_EDGE = 16

NC = 2
NS = 16
KCH = 2
ECK = E // KCH
EPS = ECK // NS
CHUNK = 200
NCHUNK = EPS // CHUNK
BLK = 4000
BPC = ECK // BLK


def _pre_body(s_ref, r_ref, w0s_ref, w0r_ref, p_ref):
    p_ref[0] = jnp.dot(s_ref[...], w0s_ref[...],
                       preferred_element_type=jnp.float32)
    p_ref[1] = jnp.dot(r_ref[...], w0r_ref[...],
                       preferred_element_type=jnp.float32)


def _precompute(sender_features, receiver_features, w0s, w0r):
    blk = 2000
    grid = (N // blk,)
    return pl.pallas_call(
        _pre_body,
        grid=grid,
        in_specs=[
            pl.BlockSpec((blk, D), lambda i: (i, 0)),
            pl.BlockSpec((blk, D), lambda i: (i, 0)),
            pl.BlockSpec((D, D), lambda i: (0, 0)),
            pl.BlockSpec((D, D), lambda i: (0, 0)),
        ],
        out_specs=pl.BlockSpec((NC, blk, D), lambda i: (0, i, 0)),
        out_shape=jax.ShapeDtypeStruct((NC, N, D), jnp.float32),
    )(sender_features, receiver_features, w0s, w0r)


NBUF = 2


def _sc_gather_body(tables_hbm, s_hbm, r_hbm, g_hbm,
                    idx_all, r0, r1, sg0, sg1, sw0, sw1):
    core = lax.axis_index("c")
    sid = lax.axis_index("s")
    base = sid * EPS
    rows = (r0, r1)
    sem_g = (sg0, sg1)
    sem_w = (sw0, sw1)

    def run_core(idx_hbm, slot):
        table = tables_hbm.at[slot]
        out = g_hbm.at[slot]
        pltpu.sync_copy(idx_hbm.at[pl.ds(base, EPS)], idx_all)

        def idx_sl(ch):
            return idx_all.at[pl.ds(ch * CHUNK, CHUNK)]

        def start(ch, b):
            pltpu.async_copy(table.at[idx_sl(ch)], rows[b], sem_g[b])

        def wait_g(b):
            pltpu.make_async_copy(table.at[idx_sl(0)], rows[b],
                                  sem_g[b]).wait()

        def wb(ch, b):
            pltpu.async_copy(rows[b],
                             out.at[pl.ds(base + ch * CHUNK, CHUNK)],
                             sem_w[b])

        def wait_w(b):
            pltpu.make_async_copy(rows[b], out.at[pl.ds(base, CHUNK)],
                                  sem_w[b]).wait()

        for b in range(NBUF):
            start(b, b)

        @pl.loop(0, NCHUNK // NBUF - 1)
        def _(i):
            ch = i * NBUF
            for b in range(NBUF):
                wait_g(b)
                wb(ch + b, b)
            for b in range(NBUF):
                wait_w(b)
                start(ch + NBUF + b, b)

        last = NCHUNK - NBUF
        for b in range(NBUF):
            wait_g(b)
            wb(last + b, b)
        for b in range(NBUF):
            wait_w(b)

    @pl.when(core == 0)
    def _():
        run_core(s_hbm, 0)

    @pl.when(core == 1)
    def _():
        run_core(r_hbm, 1)


def _sc_gather(tables, senders, receivers):
    mesh = plsc.VectorSubcoreMesh(core_axis_name="c", subcore_axis_name="s",
                                  num_cores=NC, num_subcores=NS)
    run = pl.kernel(
        _sc_gather_body,
        out_type=jax.ShapeDtypeStruct((NC, ECK, D), jnp.float32),
        mesh=mesh,
        scratch_types=(
            [pltpu.VMEM((EPS,), jnp.int32)]
            + [pltpu.VMEM((CHUNK, D), jnp.float32) for _ in range(NBUF)]
            + [pltpu.SemaphoreType.DMA for _ in range(2 * NBUF)]
        ),
    )
    return run(tables, senders, receivers)


def _mlp_body(*refs):
    gs_ref, gr_ref, ef_ref, w0e_ref, b0_ref, w1_ref, b1_ref, \
        lns_ref, lnb_ref = refs[:9]
    out_ref = refs[-1]
    z = (gs_ref[0] + gr_ref[0]
         + jnp.dot(ef_ref[...], w0e_ref[...],
                   preferred_element_type=jnp.float32)
         + b0_ref[...])
    h = jnp.maximum(z, 0.0).astype(jnp.bfloat16)
    o = jnp.dot(h, w1_ref[...],
                preferred_element_type=jnp.float32) + b1_ref[...]
    mu = jnp.mean(o, axis=-1, keepdims=True)
    d = o - mu
    var = jnp.mean(d * d, axis=-1, keepdims=True)
    out_ref[...] = d * lax.rsqrt(var + 1e-6) * lns_ref[...] + lnb_ref[...]


def _mlp_chunk(k, g, ef, w0e, b0, w1, b1, lns, lnb, buf):
    full = lambda shape: pl.BlockSpec(shape, lambda i: (0, 0))
    in_specs = [
        pl.BlockSpec((1, BLK, D), lambda i: (0, i, 0)),
        pl.BlockSpec((1, BLK, D), lambda i: (1, i, 0)),
        pl.BlockSpec((BLK, D_EDGE), lambda i: (i, 0)),
        full((D_EDGE, D)),
        full((1, D)),
        full((D, D)),
        full((1, D)),
        full((1, D)),
        full((1, D)),
    ]
    args = [g, g, ef, w0e, b0, w1, b1, lns, lnb]
    io_aliases = {}
    if buf is not None:
        in_specs.append(pl.BlockSpec(memory_space=pltpu.MemorySpace.HBM))
        args.append(buf)
        io_aliases = {9: 0}
    return pl.pallas_call(
        _mlp_body,
        grid=(BPC,),
        in_specs=in_specs,
        out_specs=pl.BlockSpec((BLK, D), lambda i, _k=k: (i + _k * BPC, 0)),
        out_shape=jax.ShapeDtypeStruct((E, D), jnp.float32),
        input_output_aliases=io_aliases,
    )(*args)


def kernel(sender_features, receiver_features, edge_features, senders,
           receivers, W0, b0, W1, b1, ln_scale, ln_bias):
    w0s = W0[:D]
    w0r = W0[D:2 * D]
    w0e = W0[2 * D:]
    senders = senders.astype(jnp.int32)
    receivers = receivers.astype(jnp.int32)
    tables = _precompute(sender_features, receiver_features, w0s, w0r)
    ef = edge_features.astype(jnp.bfloat16)
    w0e = w0e.astype(jnp.bfloat16)
    w1 = W1.astype(jnp.bfloat16)
    b0 = b0.reshape(1, D)
    b1 = b1.reshape(1, D)
    lns = ln_scale.reshape(1, D)
    lnb = ln_bias.reshape(1, D)
    buf = None
    for k in range(KCH):
        sl = slice(k * ECK, (k + 1) * ECK)
        g = _sc_gather(tables, senders[sl], receivers[sl])
        buf = _mlp_chunk(k, g, ef[sl], w0e, b0, w1, b1, lns, lnb, buf)
    return buf


# speedup vs baseline: 1.0976x; 1.0976x over previous
import jax
import jax.numpy as jnp
from jax import lax
from jax.experimental import pallas as pl
from jax.experimental.pallas import tpu as pltpu
from jax.experimental.pallas import tpu_sc as plsc

N = 10000
E = 320000
D = 128
D_EDGE = 16

NC = 2
NS = 16
KCH = 1
ECK = E // KCH
EPS = ECK // NS
CHUNK = 200
NCHUNK = EPS // CHUNK
BLK = 8000
BPC = ECK // BLK


def _pre_body(s_ref, r_ref, w0s_ref, w0r_ref, p_ref):
    p_ref[0] = jnp.dot(s_ref[...], w0s_ref[...],
                       preferred_element_type=jnp.float32)
    p_ref[1] = jnp.dot(r_ref[...], w0r_ref[...],
                       preferred_element_type=jnp.float32)


def _precompute(sender_features, receiver_features, w0s, w0r):
    blk = 2000
    grid = (N // blk,)
    return pl.pallas_call(
        _pre_body,
        grid=grid,
        in_specs=[
            pl.BlockSpec((blk, D), lambda i: (i, 0)),
            pl.BlockSpec((blk, D), lambda i: (i, 0)),
            pl.BlockSpec((D, D), lambda i: (0, 0)),
            pl.BlockSpec((D, D), lambda i: (0, 0)),
        ],
        out_specs=pl.BlockSpec((NC, blk, D), lambda i: (0, i, 0)),
        out_shape=jax.ShapeDtypeStruct((NC, N, D), jnp.float32),
    )(sender_features, receiver_features, w0s, w0r)


NBUF = 4


def _sc_gather_body(tables_hbm, s_hbm, r_hbm, g_hbm,
                    idx_all, r0, r1, r2, r3,
                    sg0, sg1, sg2, sg3, sw0, sw1, sw2, sw3):
    core = lax.axis_index("c")
    sid = lax.axis_index("s")
    base = sid * EPS
    rows = (r0, r1, r2, r3)
    sem_g = (sg0, sg1, sg2, sg3)
    sem_w = (sw0, sw1, sw2, sw3)

    def run_core(idx_hbm, slot):
        table = tables_hbm.at[slot]
        out = g_hbm.at[slot]
        pltpu.sync_copy(idx_hbm.at[pl.ds(base, EPS)], idx_all)

        def idx_sl(ch):
            return idx_all.at[pl.ds(ch * CHUNK, CHUNK)]

        def start(ch, b):
            pltpu.async_copy(table.at[idx_sl(ch)], rows[b], sem_g[b])

        def wait_g(b):
            pltpu.make_async_copy(table.at[idx_sl(0)], rows[b],
                                  sem_g[b]).wait()

        def wb(ch, b):
            pltpu.async_copy(rows[b],
                             out.at[pl.ds(base + ch * CHUNK, CHUNK)],
                             sem_w[b])

        def wait_w(b):
            pltpu.make_async_copy(rows[b], out.at[pl.ds(base, CHUNK)],
                                  sem_w[b]).wait()

        for b in range(NBUF):
            start(b, b)

        @pl.loop(0, NCHUNK // NBUF - 1)
        def _(i):
            ch = i * NBUF
            for b in range(NBUF):
                wait_g(b)
                wb(ch + b, b)
            for b in range(NBUF):
                wait_w(b)
                start(ch + NBUF + b, b)

        last = NCHUNK - NBUF
        for b in range(NBUF):
            wait_g(b)
            wb(last + b, b)
        for b in range(NBUF):
            wait_w(b)

    @pl.when(core == 0)
    def _():
        run_core(s_hbm, 0)

    @pl.when(core == 1)
    def _():
        run_core(r_hbm, 1)


def _sc_gather(tables, senders, receivers):
    mesh = plsc.VectorSubcoreMesh(core_axis_name="c", subcore_axis_name="s",
                                  num_cores=NC, num_subcores=NS)
    run = pl.kernel(
        _sc_gather_body,
        out_type=jax.ShapeDtypeStruct((NC, ECK, D), jnp.float32),
        mesh=mesh,
        scratch_types=(
            [pltpu.VMEM((EPS,), jnp.int32)]
            + [pltpu.VMEM((CHUNK, D), jnp.float32) for _ in range(NBUF)]
            + [pltpu.SemaphoreType.DMA for _ in range(2 * NBUF)]
        ),
    )
    return run(tables, senders, receivers)


def _mlp_body(*refs):
    gs_ref, gr_ref, ef_ref, w0e_ref, b0_ref, w1_ref, b1_ref, \
        lns_ref, lnb_ref = refs[:9]
    out_ref = refs[-1]
    z = (gs_ref[0] + gr_ref[0]
         + jnp.dot(ef_ref[...], w0e_ref[...],
                   preferred_element_type=jnp.float32)
         + b0_ref[...])
    h = jnp.maximum(z, 0.0).astype(jnp.bfloat16)
    o = jnp.dot(h, w1_ref[...],
                preferred_element_type=jnp.float32) + b1_ref[...]
    mu = jnp.mean(o, axis=-1, keepdims=True)
    d = o - mu
    var = jnp.mean(d * d, axis=-1, keepdims=True)
    out_ref[...] = d * lax.rsqrt(var + 1e-6) * lns_ref[...] + lnb_ref[...]


def _mlp_chunk(k, g, ef, w0e, b0, w1, b1, lns, lnb, buf):
    full = lambda shape: pl.BlockSpec(shape, lambda i: (0, 0))
    in_specs = [
        pl.BlockSpec((1, BLK, D), lambda i: (0, i, 0)),
        pl.BlockSpec((1, BLK, D), lambda i: (1, i, 0)),
        pl.BlockSpec((BLK, D_EDGE), lambda i: (i, 0)),
        full((D_EDGE, D)),
        full((1, D)),
        full((D, D)),
        full((1, D)),
        full((1, D)),
        full((1, D)),
    ]
    args = [g, g, ef, w0e, b0, w1, b1, lns, lnb]
    io_aliases = {}
    if buf is not None:
        in_specs.append(pl.BlockSpec(memory_space=pltpu.MemorySpace.HBM))
        args.append(buf)
        io_aliases = {9: 0}
    return pl.pallas_call(
        _mlp_body,
        grid=(BPC,),
        in_specs=in_specs,
        out_specs=pl.BlockSpec((BLK, D), lambda i, _k=k: (i + _k * BPC, 0)),
        out_shape=jax.ShapeDtypeStruct((E, D), jnp.float32),
        input_output_aliases=io_aliases,
    )(*args)


def kernel(sender_features, receiver_features, edge_features, senders,
           receivers, W0, b0, W1, b1, ln_scale, ln_bias):
    w0s = W0[:D]
    w0r = W0[D:2 * D]
    w0e = W0[2 * D:]
    senders = senders.astype(jnp.int32)
    receivers = receivers.astype(jnp.int32)
    tables = _precompute(sender_features, receiver_features, w0s, w0r)
    ef = edge_features.astype(jnp.bfloat16)
    w0e = w0e.astype(jnp.bfloat16)
    w1 = W1.astype(jnp.bfloat16)
    b0 = b0.reshape(1, D)
    b1 = b1.reshape(1, D)
    lns = ln_scale.reshape(1, D)
    lnb = ln_bias.reshape(1, D)
    buf = None
    for k in range(KCH):
        sl = slice(k * ECK, (k + 1) * ECK)
        g = _sc_gather(tables, senders[sl], receivers[sl])
        buf = _mlp_chunk(k, g, ef[sl], w0e, b0, w1, b1, lns, lnb, buf)
    return buf


# speedup vs baseline: 1.2570x; 1.1452x over previous
import jax
import jax.numpy as jnp
from jax import lax
from jax.experimental import pallas as pl
from jax.experimental.pallas import tpu as pltpu
from jax.experimental.pallas import tpu_sc as plsc

N = 10000
E = 320000
D = 128
D_EDGE = 16

NC = 2
NS = 16
KCH = 1
ECK = E // KCH
EPS = ECK // NS
CHUNK = 80
NCHUNK = EPS // CHUNK
NSTAGE = 10
BLK = 8000
BPC = ECK // BLK


def _pre_body(s_ref, r_ref, w0s_ref, w0r_ref, p_ref):
    p_ref[0] = jnp.dot(s_ref[...], w0s_ref[...],
                       preferred_element_type=jnp.float32)
    p_ref[1] = jnp.dot(r_ref[...], w0r_ref[...],
                       preferred_element_type=jnp.float32)


def _precompute(sender_features, receiver_features, w0s, w0r):
    blk = 2000
    grid = (N // blk,)
    return pl.pallas_call(
        _pre_body,
        grid=grid,
        in_specs=[
            pl.BlockSpec((blk, D), lambda i: (i, 0)),
            pl.BlockSpec((blk, D), lambda i: (i, 0)),
            pl.BlockSpec((D, D), lambda i: (0, 0)),
            pl.BlockSpec((D, D), lambda i: (0, 0)),
        ],
        out_specs=pl.BlockSpec((NC, blk, D), lambda i: (0, i, 0)),
        out_shape=jax.ShapeDtypeStruct((NC, N, D), jnp.float32),
    )(sender_features, receiver_features, w0s, w0r)


NBUF = 2


def _sc_gather_body(tables_hbm, s_hbm, r_hbm, g_hbm,
                    idx_all, table_sh, r0, r1,
                    sg0, sg1, sw0, sw1):
    core = lax.axis_index("c")
    sid = lax.axis_index("s")
    base = sid * EPS
    rows = (r0, r1)
    sem_g = (sg0, sg1)
    sem_w = (sw0, sw1)

    @pl.when(sid < NSTAGE)
    def _():
        nrows = N // NSTAGE
        pltpu.sync_copy(tables_hbm.at[core].at[pl.ds(sid * nrows, nrows)],
                        table_sh.at[pl.ds(sid * nrows, nrows)])
    plsc.subcore_barrier()

    def run_core(idx_hbm, slot):
        table = table_sh
        out = g_hbm.at[slot]
        pltpu.sync_copy(idx_hbm.at[pl.ds(base, EPS)], idx_all)

        def idx_sl(ch):
            return idx_all.at[pl.ds(ch * CHUNK, CHUNK)]

        def start(ch, b):
            pltpu.async_copy(table.at[idx_sl(ch)], rows[b], sem_g[b])

        def wait_g(b):
            pltpu.make_async_copy(table.at[idx_sl(0)], rows[b],
                                  sem_g[b]).wait()

        def wb(ch, b):
            pltpu.async_copy(rows[b],
                             out.at[pl.ds(base + ch * CHUNK, CHUNK)],
                             sem_w[b])

        def wait_w(b):
            pltpu.make_async_copy(rows[b], out.at[pl.ds(base, CHUNK)],
                                  sem_w[b]).wait()

        for b in range(NBUF):
            start(b, b)

        @pl.loop(0, NCHUNK // NBUF - 1)
        def _(i):
            ch = i * NBUF
            for b in range(NBUF):
                wait_g(b)
                wb(ch + b, b)
            for b in range(NBUF):
                wait_w(b)
                start(ch + NBUF + b, b)

        last = NCHUNK - NBUF
        for b in range(NBUF):
            wait_g(b)
            wb(last + b, b)
        for b in range(NBUF):
            wait_w(b)

    @pl.when(core == 0)
    def _():
        run_core(s_hbm, 0)

    @pl.when(core == 1)
    def _():
        run_core(r_hbm, 1)


def _sc_gather(tables, senders, receivers):
    mesh = plsc.VectorSubcoreMesh(core_axis_name="c", subcore_axis_name="s",
                                  num_cores=NC, num_subcores=NS)
    run = pl.kernel(
        _sc_gather_body,
        out_type=jax.ShapeDtypeStruct((NC, ECK, D), jnp.float32),
        mesh=mesh,
        scratch_types=(
            [pltpu.VMEM((EPS,), jnp.int32),
             pltpu.VMEM_SHARED((N, D), jnp.float32)]
            + [pltpu.VMEM((CHUNK, D), jnp.float32) for _ in range(NBUF)]
            + [pltpu.SemaphoreType.DMA for _ in range(2 * NBUF)]
        ),
    )
    return run(tables, senders, receivers)


def _mlp_body(*refs):
    gs_ref, gr_ref, ef_ref, w0e_ref, b0_ref, w1_ref, b1_ref, \
        lns_ref, lnb_ref = refs[:9]
    out_ref = refs[-1]
    z = (gs_ref[0] + gr_ref[0]
         + jnp.dot(ef_ref[...], w0e_ref[...],
                   preferred_element_type=jnp.float32)
         + b0_ref[...])
    h = jnp.maximum(z, 0.0).astype(jnp.bfloat16)
    o = jnp.dot(h, w1_ref[...],
                preferred_element_type=jnp.float32) + b1_ref[...]
    mu = jnp.mean(o, axis=-1, keepdims=True)
    d = o - mu
    var = jnp.mean(d * d, axis=-1, keepdims=True)
    out_ref[...] = d * lax.rsqrt(var + 1e-6) * lns_ref[...] + lnb_ref[...]


def _mlp_chunk(k, g, ef, w0e, b0, w1, b1, lns, lnb, buf):
    full = lambda shape: pl.BlockSpec(shape, lambda i: (0, 0))
    in_specs = [
        pl.BlockSpec((1, BLK, D), lambda i: (0, i, 0)),
        pl.BlockSpec((1, BLK, D), lambda i: (1, i, 0)),
        pl.BlockSpec((BLK, D_EDGE), lambda i: (i, 0)),
        full((D_EDGE, D)),
        full((1, D)),
        full((D, D)),
        full((1, D)),
        full((1, D)),
        full((1, D)),
    ]
    args = [g, g, ef, w0e, b0, w1, b1, lns, lnb]
    io_aliases = {}
    if buf is not None:
        in_specs.append(pl.BlockSpec(memory_space=pltpu.MemorySpace.HBM))
        args.append(buf)
        io_aliases = {9: 0}
    return pl.pallas_call(
        _mlp_body,
        grid=(BPC,),
        in_specs=in_specs,
        out_specs=pl.BlockSpec((BLK, D), lambda i, _k=k: (i + _k * BPC, 0)),
        out_shape=jax.ShapeDtypeStruct((E, D), jnp.float32),
        input_output_aliases=io_aliases,
    )(*args)


def kernel(sender_features, receiver_features, edge_features, senders,
           receivers, W0, b0, W1, b1, ln_scale, ln_bias):
    w0s = W0[:D]
    w0r = W0[D:2 * D]
    w0e = W0[2 * D:]
    senders = senders.astype(jnp.int32)
    receivers = receivers.astype(jnp.int32)
    tables = _precompute(sender_features, receiver_features, w0s, w0r)
    ef = edge_features.astype(jnp.bfloat16)
    w0e = w0e.astype(jnp.bfloat16)
    w1 = W1.astype(jnp.bfloat16)
    b0 = b0.reshape(1, D)
    b1 = b1.reshape(1, D)
    lns = ln_scale.reshape(1, D)
    lnb = ln_bias.reshape(1, D)
    buf = None
    for k in range(KCH):
        sl = slice(k * ECK, (k + 1) * ECK)
        g = _sc_gather(tables, senders[sl], receivers[sl])
        buf = _mlp_chunk(k, g, ef[sl], w0e, b0, w1, b1, lns, lnb, buf)
    return buf


# speedup vs baseline: 1.2636x; 1.0053x over previous
import jax
import jax.numpy as jnp
from jax import lax
from jax.experimental import pallas as pl
from jax.experimental.pallas import tpu as pltpu
from jax.experimental.pallas import tpu_sc as plsc

N = 10000
E = 320000
D = 128
D_EDGE = 16

NC = 2
NS = 16
KCH = 1
ECK = E // KCH
EPS = ECK // NS
CHUNK = 80
NCHUNK = EPS // CHUNK
NSTAGE = 10
BLK = 10000
BPC = ECK // BLK


def _pre_body(s_ref, r_ref, w0s_ref, w0r_ref, p_ref):
    p_ref[0] = jnp.dot(s_ref[...], w0s_ref[...],
                       preferred_element_type=jnp.float32)
    p_ref[1] = jnp.dot(r_ref[...], w0r_ref[...],
                       preferred_element_type=jnp.float32)


def _precompute(sender_features, receiver_features, w0s, w0r):
    blk = 2000
    grid = (N // blk,)
    return pl.pallas_call(
        _pre_body,
        grid=grid,
        in_specs=[
            pl.BlockSpec((blk, D), lambda i: (i, 0)),
            pl.BlockSpec((blk, D), lambda i: (i, 0)),
            pl.BlockSpec((D, D), lambda i: (0, 0)),
            pl.BlockSpec((D, D), lambda i: (0, 0)),
        ],
        out_specs=pl.BlockSpec((NC, blk, D), lambda i: (0, i, 0)),
        out_shape=jax.ShapeDtypeStruct((NC, N, D), jnp.float32),
    )(sender_features, receiver_features, w0s, w0r)


NBUF = 2


def _sc_gather_body(tables_hbm, s_hbm, r_hbm, g_hbm,
                    idx_all, table_sh, r0, r1,
                    sg0, sg1, sw0, sw1):
    core = lax.axis_index("c")
    sid = lax.axis_index("s")
    base = sid * EPS
    rows = (r0, r1)
    sem_g = (sg0, sg1)
    sem_w = (sw0, sw1)

    @pl.when(sid < NSTAGE)
    def _():
        nrows = N // NSTAGE
        pltpu.sync_copy(tables_hbm.at[core].at[pl.ds(sid * nrows, nrows)],
                        table_sh.at[pl.ds(sid * nrows, nrows)])
    plsc.subcore_barrier()

    def run_core(idx_hbm, slot):
        table = table_sh
        out = g_hbm.at[slot]
        pltpu.sync_copy(idx_hbm.at[pl.ds(base, EPS)], idx_all)

        def idx_sl(ch):
            return idx_all.at[pl.ds(ch * CHUNK, CHUNK)]

        def start(ch, b):
            pltpu.async_copy(table.at[idx_sl(ch)], rows[b], sem_g[b])

        def wait_g(b):
            pltpu.make_async_copy(table.at[idx_sl(0)], rows[b],
                                  sem_g[b]).wait()

        def wb(ch, b):
            pltpu.async_copy(rows[b],
                             out.at[pl.ds(base + ch * CHUNK, CHUNK)],
                             sem_w[b])

        def wait_w(b):
            pltpu.make_async_copy(rows[b], out.at[pl.ds(base, CHUNK)],
                                  sem_w[b]).wait()

        for b in range(NBUF):
            start(b, b)

        @pl.loop(0, NCHUNK // NBUF - 1)
        def _(i):
            ch = i * NBUF
            for b in range(NBUF):
                wait_g(b)
                wb(ch + b, b)
            for b in range(NBUF):
                wait_w(b)
                start(ch + NBUF + b, b)

        last = NCHUNK - NBUF
        for b in range(NBUF):
            wait_g(b)
            wb(last + b, b)
        for b in range(NBUF):
            wait_w(b)

    @pl.when(core == 0)
    def _():
        run_core(s_hbm, 0)

    @pl.when(core == 1)
    def _():
        run_core(r_hbm, 1)


def _sc_gather(tables, senders, receivers):
    mesh = plsc.VectorSubcoreMesh(core_axis_name="c", subcore_axis_name="s",
                                  num_cores=NC, num_subcores=NS)
    run = pl.kernel(
        _sc_gather_body,
        out_type=jax.ShapeDtypeStruct((NC, ECK, D), jnp.float32),
        mesh=mesh,
        scratch_types=(
            [pltpu.VMEM((EPS,), jnp.int32),
             pltpu.VMEM_SHARED((N, D), jnp.float32)]
            + [pltpu.VMEM((CHUNK, D), jnp.float32) for _ in range(NBUF)]
            + [pltpu.SemaphoreType.DMA for _ in range(2 * NBUF)]
        ),
    )
    return run(tables, senders, receivers)


def _mlp_body(*refs):
    gs_ref, gr_ref, ef_ref, w0e_ref, b0_ref, w1_ref, b1_ref, \
        lns_ref, lnb_ref = refs[:9]
    out_ref = refs[-1]
    z = (gs_ref[0] + gr_ref[0]
         + jnp.dot(ef_ref[...], w0e_ref[...],
                   preferred_element_type=jnp.float32)
         + b0_ref[...])
    h = jnp.maximum(z, 0.0).astype(jnp.bfloat16)
    o = jnp.dot(h, w1_ref[...],
                preferred_element_type=jnp.float32) + b1_ref[...]
    mu = jnp.mean(o, axis=-1, keepdims=True)
    d = o - mu
    var = jnp.mean(d * d, axis=-1, keepdims=True)
    out_ref[...] = d * lax.rsqrt(var + 1e-6) * lns_ref[...] + lnb_ref[...]


def _mlp_chunk(k, g, ef, w0e, b0, w1, b1, lns, lnb, buf):
    full = lambda shape: pl.BlockSpec(shape, lambda i: (0, 0))
    in_specs = [
        pl.BlockSpec((1, BLK, D), lambda i: (0, i, 0)),
        pl.BlockSpec((1, BLK, D), lambda i: (1, i, 0)),
        pl.BlockSpec((BLK, D_EDGE), lambda i: (i, 0)),
        full((D_EDGE, D)),
        full((1, D)),
        full((D, D)),
        full((1, D)),
        full((1, D)),
        full((1, D)),
    ]
    args = [g, g, ef, w0e, b0, w1, b1, lns, lnb]
    io_aliases = {}
    if buf is not None:
        in_specs.append(pl.BlockSpec(memory_space=pltpu.MemorySpace.HBM))
        args.append(buf)
        io_aliases = {9: 0}
    return pl.pallas_call(
        _mlp_body,
        grid=(BPC,),
        in_specs=in_specs,
        out_specs=pl.BlockSpec((BLK, D), lambda i, _k=k: (i + _k * BPC, 0)),
        out_shape=jax.ShapeDtypeStruct((E, D), jnp.float32),
        input_output_aliases=io_aliases,
    )(*args)


def kernel(sender_features, receiver_features, edge_features, senders,
           receivers, W0, b0, W1, b1, ln_scale, ln_bias):
    w0s = W0[:D]
    w0r = W0[D:2 * D]
    w0e = W0[2 * D:]
    senders = senders.astype(jnp.int32)
    receivers = receivers.astype(jnp.int32)
    tables = _precompute(sender_features, receiver_features, w0s, w0r)
    ef = edge_features.astype(jnp.bfloat16)
    w0e = w0e.astype(jnp.bfloat16)
    w1 = W1.astype(jnp.bfloat16)
    b0 = b0.reshape(1, D)
    b1 = b1.reshape(1, D)
    lns = ln_scale.reshape(1, D)
    lnb = ln_bias.reshape(1, D)
    buf = None
    for k in range(KCH):
        sl = slice(k * ECK, (k + 1) * ECK)
        g = _sc_gather(tables, senders[sl], receivers[sl])
        buf = _mlp_chunk(k, g, ef[sl], w0e, b0, w1, b1, lns, lnb, buf)
    return buf


# speedup vs baseline: 1.4416x; 1.1409x over previous
import jax
import jax.numpy as jnp
from jax import lax
from jax.experimental import pallas as pl
from jax.experimental.pallas import tpu as pltpu
from jax.experimental.pallas import tpu_sc as plsc

N = 10000
E = 320000
D = 128
D_EDGE = 16

NC = 2
NS = 16
KCH = 1
ECK = E // KCH
EPS = ECK // NS
CHUNK = 80
NCHUNK = EPS // CHUNK
NSTAGE = 10
BLK = 10000
BPC = ECK // BLK


def _pre_body(s_ref, r_ref, w0s_ref, w0r_ref, p_ref):
    p_ref[0] = jnp.dot(s_ref[...], w0s_ref[...],
                       preferred_element_type=jnp.float32)
    p_ref[1] = jnp.dot(r_ref[...], w0r_ref[...],
                       preferred_element_type=jnp.float32)


def _precompute(sender_features, receiver_features, w0s, w0r):
    blk = 2000
    grid = (N // blk,)
    return pl.pallas_call(
        _pre_body,
        grid=grid,
        in_specs=[
            pl.BlockSpec((blk, D), lambda i: (i, 0)),
            pl.BlockSpec((blk, D), lambda i: (i, 0)),
            pl.BlockSpec((D, D), lambda i: (0, 0)),
            pl.BlockSpec((D, D), lambda i: (0, 0)),
        ],
        out_specs=pl.BlockSpec((NC, blk, D), lambda i: (0, i, 0)),
        out_shape=jax.ShapeDtypeStruct((NC, N, D), jnp.float32),
    )(sender_features, receiver_features, w0s, w0r)


NBUF = 3


def _sc_gather_body(tables_hbm, s_hbm, r_hbm, g_hbm,
                    idx_all, table_sh, r0, r1, r2,
                    sg0, sg1, sg2, sw0, sw1, sw2):
    core = lax.axis_index("c")
    sid = lax.axis_index("s")
    base = sid * EPS
    rows = (r0, r1, r2)
    sem_g = (sg0, sg1, sg2)
    sem_w = (sw0, sw1, sw2)

    @pl.when(sid < NSTAGE)
    def _():
        nrows = N // NSTAGE
        pltpu.sync_copy(tables_hbm.at[core].at[pl.ds(sid * nrows, nrows)],
                        table_sh.at[pl.ds(sid * nrows, nrows)])
    plsc.subcore_barrier()

    def run_core(idx_hbm, slot):
        table = table_sh
        out = g_hbm.at[slot]
        pltpu.sync_copy(idx_hbm.at[pl.ds(base, EPS)], idx_all)

        def idx_sl(ch):
            return idx_all.at[pl.ds(ch * CHUNK, CHUNK)]

        def start(ch, b):
            pltpu.async_copy(table.at[idx_sl(ch)], rows[b], sem_g[b])

        def wait_g(b):
            pltpu.make_async_copy(table.at[idx_sl(0)], rows[b],
                                  sem_g[b]).wait()

        def wb(ch, b):
            pltpu.async_copy(rows[b],
                             out.at[pl.ds(base + ch * CHUNK, CHUNK)],
                             sem_w[b])

        def wait_w(b):
            pltpu.make_async_copy(rows[b], out.at[pl.ds(base, CHUNK)],
                                  sem_w[b]).wait()

        for b in range(NBUF):
            start(b, b)

        @pl.loop(0, NCHUNK // NBUF)
        def _(i):
            ch = i * NBUF
            for b in range(NBUF):
                wait_g(b)
                wb(ch + b, b)
            for b in range(NBUF):
                wait_w(b)

                @pl.when(ch + NBUF + b < NCHUNK)
                def _(b=b):
                    start(ch + NBUF + b, b)

        rem = NCHUNK - (NCHUNK // NBUF) * NBUF
        for b in range(rem):
            wait_g(b)
            wb(NCHUNK - rem + b, b)
        for b in range(rem):
            wait_w(b)

    @pl.when(core == 0)
    def _():
        run_core(s_hbm, 0)

    @pl.when(core == 1)
    def _():
        run_core(r_hbm, 1)


def _sc_gather(tables, senders, receivers):
    mesh = plsc.VectorSubcoreMesh(core_axis_name="c", subcore_axis_name="s",
                                  num_cores=NC, num_subcores=NS)
    run = pl.kernel(
        _sc_gather_body,
        out_type=jax.ShapeDtypeStruct((NC, ECK, D), jnp.float32),
        mesh=mesh,
        scratch_types=(
            [pltpu.VMEM((EPS,), jnp.int32),
             pltpu.VMEM_SHARED((N, D), jnp.float32)]
            + [pltpu.VMEM((CHUNK, D), jnp.float32) for _ in range(NBUF)]
            + [pltpu.SemaphoreType.DMA for _ in range(2 * NBUF)]
        ),
    )
    return run(tables, senders, receivers)


def _mlp_body(*refs):
    gs_ref, gr_ref, ef_ref, w0e_ref, b0_ref, w1_ref, b1_ref, \
        lns_ref, lnb_ref = refs[:9]
    out_ref = refs[-1]
    z = (gs_ref[0] + gr_ref[0]
         + jnp.dot(ef_ref[...], w0e_ref[...],
                   preferred_element_type=jnp.float32)
         + b0_ref[...])
    h = jnp.maximum(z, 0.0).astype(jnp.bfloat16)
    o = jnp.dot(h, w1_ref[...],
                preferred_element_type=jnp.float32) + b1_ref[...]
    mu = jnp.mean(o, axis=-1, keepdims=True)
    d = o - mu
    var = jnp.mean(d * d, axis=-1, keepdims=True)
    out_ref[...] = d * lax.rsqrt(var + 1e-6) * lns_ref[...] + lnb_ref[...]


def _mlp_chunk(k, g, ef, w0e, b0, w1, b1, lns, lnb, buf):
    full = lambda shape: pl.BlockSpec(shape, lambda i: (0, 0))
    in_specs = [
        pl.BlockSpec((1, BLK, D), lambda i: (0, i, 0)),
        pl.BlockSpec((1, BLK, D), lambda i: (1, i, 0)),
        pl.BlockSpec((BLK, D_EDGE), lambda i: (i, 0)),
        full((D_EDGE, D)),
        full((1, D)),
        full((D, D)),
        full((1, D)),
        full((1, D)),
        full((1, D)),
    ]
    args = [g, g, ef, w0e, b0, w1, b1, lns, lnb]
    io_aliases = {}
    if buf is not None:
        in_specs.append(pl.BlockSpec(memory_space=pltpu.MemorySpace.HBM))
        args.append(buf)
        io_aliases = {9: 0}
    return pl.pallas_call(
        _mlp_body,
        grid=(BPC,),
        in_specs=in_specs,
        out_specs=pl.BlockSpec((BLK, D), lambda i, _k=k: (i + _k * BPC, 0)),
        out_shape=jax.ShapeDtypeStruct((E, D), jnp.float32),
        input_output_aliases=io_aliases,
    )(*args)


def kernel(sender_features, receiver_features, edge_features, senders,
           receivers, W0, b0, W1, b1, ln_scale, ln_bias):
    w0s = W0[:D]
    w0r = W0[D:2 * D]
    w0e = W0[2 * D:]
    senders = senders.astype(jnp.int32)
    receivers = receivers.astype(jnp.int32)
    tables = _precompute(sender_features, receiver_features, w0s, w0r)
    ef = edge_features.astype(jnp.bfloat16)
    w0e = w0e.astype(jnp.bfloat16)
    w1 = W1.astype(jnp.bfloat16)
    b0 = b0.reshape(1, D)
    b1 = b1.reshape(1, D)
    lns = ln_scale.reshape(1, D)
    lnb = ln_bias.reshape(1, D)
    buf = None
    for k in range(KCH):
        sl = slice(k * ECK, (k + 1) * ECK)
        g = _sc_gather(tables, senders[sl], receivers[sl])
        buf = _mlp_chunk(k, g, ef[sl], w0e, b0, w1, b1, lns, lnb, buf)
    return buf
